# baseline (speedup 1.0000x reference)
import functools

import jax
import jax.numpy as jnp
from jax import lax
from jax.experimental import pallas as pl
from jax.experimental.pallas import tpu as pltpu
from jax.experimental.pallas import tpu_sc as plsc

ROWS = 128
COLS = 32768
K = 64

_NC = 2
_NS = 16
_L = 16
_NW = _NC * _NS
_CHUNKS_PER_ROW = 4
_SC_CHUNK = COLS // _CHUNKS_PER_ROW
_ROWS_PER_W = ROWS // _NW
_NCH = _ROWS_PER_W * _CHUNKS_PER_ROW


def _sc_reduce_body(out_hbm, y_hbm, res_hbm, ob0, ob1, yb0, yb1, resbuf,
                    so0, so1, sy0, sy1):
    w = lax.axis_index("s") * _NC + lax.axis_index("c")
    obufs = (ob0, ob1)
    ybufs = (yb0, yb1)
    osems = (so0, so1)
    ysems = (sy0, sy1)

    def start(j):
        b = j % 2
        oc = pltpu.async_copy(out_hbm.at[w, j], obufs[b], osems[b])
        yc = pltpu.async_copy(y_hbm.at[w, j], ybufs[b], ysems[b])
        return oc, yc

    inflight = start(0)
    acc = jnp.zeros((_L,), jnp.float32)
    for j in range(_NCH):
        oc, yc = inflight
        if j + 1 < _NCH:
            nxt = start(j + 1)
        oc.wait()
        yc.wait()
        if j + 1 < _NCH:
            inflight = nxt
        b = j % 2
        ob, yb = obufs[b], ybufs[b]

        def body(i, a):
            o = ob[pl.ds(i * _L, _L)]
            yv = yb[pl.ds(i * _L, _L)]
            d = o - yv
            return a + d * d

        acc = lax.fori_loop(0, _SC_CHUNK // _L, body, acc)
        if j % _CHUNKS_PER_ROW == _CHUNKS_PER_ROW - 1:
            resbuf[j // _CHUNKS_PER_ROW, :] = acc
            acc = jnp.zeros((_L,), jnp.float32)
    pltpu.sync_copy(resbuf, res_hbm.at[w])


_sc_mesh = plsc.VectorSubcoreMesh(core_axis_name="c", subcore_axis_name="s")

_sc_reduce = pl.kernel(
    _sc_reduce_body,
    mesh=_sc_mesh,
    out_type=jax.ShapeDtypeStruct((_NW, _ROWS_PER_W, _L), jnp.float32),
    scratch_types=[
        pltpu.VMEM((_SC_CHUNK,), jnp.float32),
        pltpu.VMEM((_SC_CHUNK,), jnp.float32),
        pltpu.VMEM((_SC_CHUNK,), jnp.float32),
        pltpu.VMEM((_SC_CHUNK,), jnp.float32),
        pltpu.VMEM((_ROWS_PER_W, _L), jnp.float32),
        pltpu.SemaphoreType.DMA,
        pltpu.SemaphoreType.DMA,
        pltpu.SemaphoreType.DMA,
        pltpu.SemaphoreType.DMA,
    ],
)


def _topk_body(v_ref, res_ref):
    vr = v_ref[...]
    s = jnp.sum(vr, axis=1, keepdims=True) * (1.0 / COLS)
    v = s.reshape(1, ROWS)
    gt = v > v.reshape(ROWS, 1)
    rank = jnp.sum(gt.astype(jnp.float32), axis=1).reshape(1, ROWS)
    cand = rank < K
    t = jnp.min(jnp.where(cand, v, jnp.inf))
    above = v > t
    n_above = jnp.sum(above.astype(jnp.float32))
    total = jnp.sum(jnp.where(above, v, 0.0)) + t * (K - n_above)
    res_ref[...] = total.reshape(1, 1)


def _topk_sum(partials):
    res = pl.pallas_call(
        _topk_body,
        out_shape=jax.ShapeDtypeStruct((1, 1), jnp.float32),
    )(partials.reshape(ROWS, _L))
    return res[0, 0]


def kernel(out, y):
    out3 = out.reshape(_NW, _NCH, _SC_CHUNK)
    y3 = y.reshape(_NW, _NCH, _SC_CHUNK)
    res = _sc_reduce(out3, y3)
    return _topk_sum(res)


# speedup vs baseline: 1.1767x; 1.1767x over previous
import functools

import jax
import jax.numpy as jnp
from jax import lax
from jax.experimental import pallas as pl
from jax.experimental.pallas import tpu as pltpu
from jax.experimental.pallas import tpu_sc as plsc

ROWS = 128
COLS = 32768
K = 64

_NC = 2
_NS = 16
_L = 16
_NW = _NC * _NS
_CHUNKS_PER_ROW = 4
_SC_CHUNK = COLS // _CHUNKS_PER_ROW
_ROWS_PER_W = ROWS // _NW
_NCH = _ROWS_PER_W * _CHUNKS_PER_ROW
_UNROLL = 8


def _sc_reduce_body(out_hbm, y_hbm, res_hbm, ob0, ob1, yb0, yb1, resbuf,
                    so0, so1, sy0, sy1):
    w = lax.axis_index("s") * _NC + lax.axis_index("c")
    obufs = (ob0, ob1)
    ybufs = (yb0, yb1)
    osems = (so0, so1)
    ysems = (sy0, sy1)

    def start(j):
        b = j % 2
        oc = pltpu.async_copy(out_hbm.at[w, j], obufs[b], osems[b])
        yc = pltpu.async_copy(y_hbm.at[w, j], ybufs[b], ysems[b])
        return oc, yc

    inflight = start(0)
    accs = tuple(jnp.zeros((_L,), jnp.float32) for _ in range(_UNROLL))
    for j in range(_NCH):
        oc, yc = inflight
        if j + 1 < _NCH:
            nxt = start(j + 1)
        oc.wait()
        yc.wait()
        if j + 1 < _NCH:
            inflight = nxt
        b = j % 2
        ob, yb = obufs[b], ybufs[b]

        def body(i, accs):
            base = i * (_L * _UNROLL)
            out_accs = []
            for u in range(_UNROLL):
                o = ob[pl.ds(base + u * _L, _L)]
                yv = yb[pl.ds(base + u * _L, _L)]
                d = o - yv
                out_accs.append(accs[u] + d * d)
            return tuple(out_accs)

        accs = lax.fori_loop(0, _SC_CHUNK // (_L * _UNROLL), body, accs)
        if j % _CHUNKS_PER_ROW == _CHUNKS_PER_ROW - 1:
            total = accs[0]
            for u in range(1, _UNROLL):
                total = total + accs[u]
            resbuf[j // _CHUNKS_PER_ROW, :] = total
            accs = tuple(jnp.zeros((_L,), jnp.float32) for _ in range(_UNROLL))
    pltpu.sync_copy(resbuf, res_hbm.at[w])


_sc_mesh = plsc.VectorSubcoreMesh(core_axis_name="c", subcore_axis_name="s")

_sc_reduce = pl.kernel(
    _sc_reduce_body,
    mesh=_sc_mesh,
    out_type=jax.ShapeDtypeStruct((_NW, _ROWS_PER_W, _L), jnp.float32),
    scratch_types=[
        pltpu.VMEM((_SC_CHUNK,), jnp.float32),
        pltpu.VMEM((_SC_CHUNK,), jnp.float32),
        pltpu.VMEM((_SC_CHUNK,), jnp.float32),
        pltpu.VMEM((_SC_CHUNK,), jnp.float32),
        pltpu.VMEM((_ROWS_PER_W, _L), jnp.float32),
        pltpu.SemaphoreType.DMA,
        pltpu.SemaphoreType.DMA,
        pltpu.SemaphoreType.DMA,
        pltpu.SemaphoreType.DMA,
    ],
)


def _topk_body(v_ref, res_ref):
    vr = v_ref[...]
    s = jnp.sum(vr, axis=1, keepdims=True) * (1.0 / COLS)
    v = s.reshape(1, ROWS)
    gt = v > v.reshape(ROWS, 1)
    rank = jnp.sum(gt.astype(jnp.float32), axis=1).reshape(1, ROWS)
    cand = rank < K
    t = jnp.min(jnp.where(cand, v, jnp.inf))
    above = v > t
    n_above = jnp.sum(above.astype(jnp.float32))
    total = jnp.sum(jnp.where(above, v, 0.0)) + t * (K - n_above)
    res_ref[...] = total.reshape(1, 1)


def _topk_sum(partials):
    res = pl.pallas_call(
        _topk_body,
        out_shape=jax.ShapeDtypeStruct((1, 1), jnp.float32),
    )(partials.reshape(ROWS, _L))
    return res[0, 0]


def kernel(out, y):
    out3 = out.reshape(_NW, _NCH, _SC_CHUNK)
    y3 = y.reshape(_NW, _NCH, _SC_CHUNK)
    res = _sc_reduce(out3, y3)
    return _topk_sum(res)


# speedup vs baseline: 2.1406x; 1.8193x over previous
import functools

import jax
import jax.numpy as jnp
from jax import lax
from jax.experimental import pallas as pl
from jax.experimental.pallas import tpu as pltpu
from jax.experimental.pallas import tpu_sc as plsc

ROWS = 128
COLS = 32768
K = 64

_NC = 2
_NS = 16
_L = 16
_NW = _NC * _NS
_CHUNKS_PER_ROW = 4
_SC_CHUNK = COLS // _CHUNKS_PER_ROW
_ROWS_PER_W = ROWS // _NW
_NCH = _ROWS_PER_W * _CHUNKS_PER_ROW
_UNROLL = 8


def _sc_reduce_body(out_hbm, y_hbm, res_hbm, ob0, ob1, yb0, yb1, resbuf,
                    so0, so1, sy0, sy1):
    w = lax.axis_index("s") * _NC + lax.axis_index("c")
    obufs = (ob0, ob1)
    ybufs = (yb0, yb1)
    osems = (so0, so1)
    ysems = (sy0, sy1)

    def start(j):
        b = j % 2
        row = w * _ROWS_PER_W + j // _CHUNKS_PER_ROW
        col = (j % _CHUNKS_PER_ROW) * _SC_CHUNK
        oc = pltpu.async_copy(out_hbm.at[row, pl.ds(col, _SC_CHUNK)],
                              obufs[b], osems[b])
        yc = pltpu.async_copy(y_hbm.at[row, pl.ds(col, _SC_CHUNK)],
                              ybufs[b], ysems[b])
        return oc, yc

    inflight = start(0)
    accs = tuple(jnp.zeros((_L,), jnp.float32) for _ in range(_UNROLL))
    for j in range(_NCH):
        oc, yc = inflight
        if j + 1 < _NCH:
            nxt = start(j + 1)
        oc.wait()
        yc.wait()
        if j + 1 < _NCH:
            inflight = nxt
        b = j % 2
        ob, yb = obufs[b], ybufs[b]

        def body(i, accs):
            base = i * (_L * _UNROLL)
            out_accs = []
            for u in range(_UNROLL):
                o = ob[pl.ds(base + u * _L, _L)]
                yv = yb[pl.ds(base + u * _L, _L)]
                d = o - yv
                out_accs.append(accs[u] + d * d)
            return tuple(out_accs)

        accs = lax.fori_loop(0, _SC_CHUNK // (_L * _UNROLL), body, accs)
        if j % _CHUNKS_PER_ROW == _CHUNKS_PER_ROW - 1:
            total = accs[0]
            for u in range(1, _UNROLL):
                total = total + accs[u]
            resbuf[j // _CHUNKS_PER_ROW, :] = total
            accs = tuple(jnp.zeros((_L,), jnp.float32) for _ in range(_UNROLL))
    pltpu.sync_copy(resbuf, res_hbm.at[w])


_sc_mesh = plsc.VectorSubcoreMesh(core_axis_name="c", subcore_axis_name="s")

_sc_reduce = pl.kernel(
    _sc_reduce_body,
    mesh=_sc_mesh,
    out_type=jax.ShapeDtypeStruct((_NW, _ROWS_PER_W, _L), jnp.float32),
    scratch_types=[
        pltpu.VMEM((_SC_CHUNK,), jnp.float32),
        pltpu.VMEM((_SC_CHUNK,), jnp.float32),
        pltpu.VMEM((_SC_CHUNK,), jnp.float32),
        pltpu.VMEM((_SC_CHUNK,), jnp.float32),
        pltpu.VMEM((_ROWS_PER_W, _L), jnp.float32),
        pltpu.SemaphoreType.DMA,
        pltpu.SemaphoreType.DMA,
        pltpu.SemaphoreType.DMA,
        pltpu.SemaphoreType.DMA,
    ],
)


def _topk_body(v_ref, res_ref):
    vr = v_ref[...]
    s = jnp.sum(vr, axis=1, keepdims=True) * (1.0 / COLS)
    v = s.reshape(1, ROWS)
    gt = v > v.reshape(ROWS, 1)
    rank = jnp.sum(gt.astype(jnp.float32), axis=1).reshape(1, ROWS)
    cand = rank < K
    t = jnp.min(jnp.where(cand, v, jnp.inf))
    above = v > t
    n_above = jnp.sum(above.astype(jnp.float32))
    total = jnp.sum(jnp.where(above, v, 0.0)) + t * (K - n_above)
    res_ref[...] = total.reshape(1, 1)


def _topk_sum(partials):
    res = pl.pallas_call(
        _topk_body,
        out_shape=jax.ShapeDtypeStruct((1, 1), jnp.float32),
    )(partials.reshape(ROWS, _L))
    return res[0, 0]


def kernel(out, y):
    res = _sc_reduce(out, y)
    return _topk_sum(res)


# speedup vs baseline: 2.6245x; 1.2260x over previous
import jax
import jax.numpy as jnp
from jax import lax
from jax.experimental import pallas as pl
from jax.experimental.pallas import tpu as pltpu
from jax.experimental.pallas import tpu_sc as plsc

ROWS = 128
COLS = 32768
K = 64

_NC = 2
_NS = 16
_L = 16
_NW = _NC * _NS

SC_ROWS = 32
TC_ROWS = ROWS - SC_ROWS
_SC_RPW = SC_ROWS // _NW
_SC_CHUNK = 8192
_SC_CPR = COLS // _SC_CHUNK
_SC_NCH = _SC_RPW * _SC_CPR
_UNROLL = 8
TC_CHUNK = 4096


def _sc_reduce_body(out_hbm, y_hbm, res_hbm, ob0, ob1, yb0, yb1, resbuf,
                    so0, so1, sy0, sy1):
    w = lax.axis_index("s") * _NC + lax.axis_index("c")
    obufs = (ob0, ob1)
    ybufs = (yb0, yb1)
    osems = (so0, so1)
    ysems = (sy0, sy1)

    def start(j):
        b = j % 2
        row = TC_ROWS + w * _SC_RPW + j // _SC_CPR
        col = (j % _SC_CPR) * _SC_CHUNK
        oc = pltpu.async_copy(out_hbm.at[row, pl.ds(col, _SC_CHUNK)],
                              obufs[b], osems[b])
        yc = pltpu.async_copy(y_hbm.at[row, pl.ds(col, _SC_CHUNK)],
                              ybufs[b], ysems[b])
        return oc, yc

    inflight = start(0)
    accs = tuple(jnp.zeros((_L,), jnp.float32) for _ in range(_UNROLL))
    for j in range(_SC_NCH):
        oc, yc = inflight
        if j + 1 < _SC_NCH:
            nxt = start(j + 1)
        oc.wait()
        yc.wait()
        if j + 1 < _SC_NCH:
            inflight = nxt
        b = j % 2
        ob, yb = obufs[b], ybufs[b]

        def body(i, a):
            base = i * (_L * _UNROLL)
            upd = []
            for u in range(_UNROLL):
                o = ob[pl.ds(base + u * _L, _L)]
                yv = yb[pl.ds(base + u * _L, _L)]
                d = o - yv
                upd.append(a[u] + d * d)
            return tuple(upd)

        accs = lax.fori_loop(0, _SC_CHUNK // (_L * _UNROLL), body, accs)
        if j % _SC_CPR == _SC_CPR - 1:
            total = accs[0]
            for u in range(1, _UNROLL):
                total = total + accs[u]
            resbuf[j // _SC_CPR, :] = total
            accs = tuple(jnp.zeros((_L,), jnp.float32) for _ in range(_UNROLL))
    pltpu.sync_copy(resbuf, res_hbm.at[w])


_sc_reduce_cache = []


def _sc_reduce(out, y):
    if not _sc_reduce_cache:
        mesh = plsc.VectorSubcoreMesh(core_axis_name="c", subcore_axis_name="s")
        _sc_reduce_cache.append(pl.kernel(
            _sc_reduce_body,
            mesh=mesh,
            out_type=jax.ShapeDtypeStruct((_NW, _SC_RPW, _L), jnp.float32),
            scratch_types=[
                pltpu.VMEM((_SC_CHUNK,), jnp.float32),
                pltpu.VMEM((_SC_CHUNK,), jnp.float32),
                pltpu.VMEM((_SC_CHUNK,), jnp.float32),
                pltpu.VMEM((_SC_CHUNK,), jnp.float32),
                pltpu.VMEM((_SC_RPW, _L), jnp.float32),
                pltpu.SemaphoreType.DMA,
                pltpu.SemaphoreType.DMA,
                pltpu.SemaphoreType.DMA,
                pltpu.SemaphoreType.DMA,
            ],
        ))
    return _sc_reduce_cache[0](out, y)


def _tc_reduce_body(out_ref, y_ref, res_ref):
    pid = pl.program_id(0)
    d = out_ref[...] - y_ref[...]
    partial = jnp.sum(d * d, axis=1, keepdims=True)

    @pl.when(pid == 0)
    def _init():
        res_ref[...] = partial

    @pl.when(pid != 0)
    def _accum():
        res_ref[...] += partial


def _tc_reduce(out, y):
    nsteps = COLS // TC_CHUNK
    return pl.pallas_call(
        _tc_reduce_body,
        grid=(nsteps,),
        in_specs=[
            pl.BlockSpec((TC_ROWS, TC_CHUNK), lambda i: (0, i)),
            pl.BlockSpec((TC_ROWS, TC_CHUNK), lambda i: (0, i)),
        ],
        out_specs=pl.BlockSpec((TC_ROWS, 1), lambda i: (0, 0)),
        out_shape=jax.ShapeDtypeStruct((TC_ROWS, 1), jnp.float32),
        compiler_params=pltpu.CompilerParams(
            dimension_semantics=("arbitrary",),
        ),
    )(out, y)


def _merge_body(tc_ref, sc_ref, res_ref):
    vtc = tc_ref[...]
    vsc = jnp.sum(sc_ref[...], axis=1, keepdims=True)
    v = jnp.concatenate([vtc, vsc], axis=0) * (1.0 / COLS)
    keys = lax.bitcast_convert_type(v, jnp.int32)

    def bisect(_, carry):
        lo, hi = carry
        mid = lo + (hi - lo) // 2
        cnt = jnp.sum((keys >= mid).astype(jnp.int32))
        big = cnt >= K
        return (jnp.where(big, mid, lo), jnp.where(big, hi, mid))

    lo, hi = lax.fori_loop(0, 31, bisect, (jnp.int32(0), jnp.int32(2**31 - 1)))
    above = keys > lo
    n_above = jnp.sum(above.astype(jnp.float32))
    s_above = jnp.sum(jnp.where(above, v, 0.0))
    t = jnp.max(jnp.where(keys == lo, v, -1.0))
    total = s_above + t * (K - n_above)
    res_ref[...] = total.reshape(1, 1)


def _merge(tc_part, sc_part):
    res = pl.pallas_call(
        _merge_body,
        out_shape=jax.ShapeDtypeStruct((1, 1), jnp.float32),
    )(tc_part, sc_part.reshape(SC_ROWS, _L))
    return res[0, 0]


def kernel(out, y):
    sc_part = _sc_reduce(out, y)
    tc_part = _tc_reduce(out, y)
    return _merge(tc_part, sc_part)


# speedup vs baseline: 2.9225x; 1.1136x over previous
import jax
import jax.numpy as jnp
from jax import lax
from jax.experimental import pallas as pl
from jax.experimental.pallas import tpu as pltpu
from jax.experimental.pallas import tpu_sc as plsc

ROWS = 128
COLS = 32768
K = 64

_NC = 2
_NS = 16
_L = 16
_NW = _NC * _NS

SC_ROWS = 32
TC_ROWS = ROWS - SC_ROWS
_SC_RPW = SC_ROWS // _NW
_SC_CHUNK = 8192
_SC_CPR = COLS // _SC_CHUNK
_SC_NCH = _SC_RPW * _SC_CPR
_UNROLL = 16
_NBUF = 4
TC_CHUNK = 4096


def _sc_reduce_body(out_hbm, y_hbm, res_hbm,
                    ob0, ob1, ob2, ob3, yb0, yb1, yb2, yb3, resbuf,
                    so0, so1, so2, so3, sy0, sy1, sy2, sy3):
    w = lax.axis_index("s") * _NC + lax.axis_index("c")
    obufs = (ob0, ob1, ob2, ob3)
    ybufs = (yb0, yb1, yb2, yb3)
    osems = (so0, so1, so2, so3)
    ysems = (sy0, sy1, sy2, sy3)

    def start(j):
        b = j % _NBUF
        row = TC_ROWS + w * _SC_RPW + j // _SC_CPR
        col = (j % _SC_CPR) * _SC_CHUNK
        oc = pltpu.async_copy(out_hbm.at[row, pl.ds(col, _SC_CHUNK)],
                              obufs[b], osems[b])
        yc = pltpu.async_copy(y_hbm.at[row, pl.ds(col, _SC_CHUNK)],
                              ybufs[b], ysems[b])
        return oc, yc

    pending = {}
    for j in range(min(_NBUF, _SC_NCH)):
        pending[j] = start(j)
    accs = tuple(jnp.zeros((_L,), jnp.float32) for _ in range(_UNROLL))
    for j in range(_SC_NCH):
        oc, yc = pending.pop(j)
        oc.wait()
        yc.wait()
        b = j % _NBUF
        ob, yb = obufs[b], ybufs[b]

        def body(i, a):
            base = i * (_L * _UNROLL)
            upd = []
            for u in range(_UNROLL):
                o = ob[pl.ds(base + u * _L, _L)]
                yv = yb[pl.ds(base + u * _L, _L)]
                d = o - yv
                upd.append(a[u] + d * d)
            return tuple(upd)

        accs = lax.fori_loop(0, _SC_CHUNK // (_L * _UNROLL), body, accs)
        if j + _NBUF < _SC_NCH:
            pending[j + _NBUF] = start(j + _NBUF)
        if j % _SC_CPR == _SC_CPR - 1:
            total = accs[0]
            for u in range(1, _UNROLL):
                total = total + accs[u]
            resbuf[j // _SC_CPR, :] = total
            accs = tuple(jnp.zeros((_L,), jnp.float32) for _ in range(_UNROLL))
    pltpu.sync_copy(resbuf, res_hbm.at[w])


_sc_reduce_cache = []


def _sc_reduce(out, y):
    if not _sc_reduce_cache:
        mesh = plsc.VectorSubcoreMesh(core_axis_name="c", subcore_axis_name="s")
        _sc_reduce_cache.append(pl.kernel(
            _sc_reduce_body,
            mesh=mesh,
            out_type=jax.ShapeDtypeStruct((_NW, _SC_RPW, _L), jnp.float32),
            scratch_types=(
                [pltpu.VMEM((_SC_CHUNK,), jnp.float32)] * (2 * _NBUF)
                + [pltpu.VMEM((_SC_RPW, _L), jnp.float32)]
                + [pltpu.SemaphoreType.DMA] * (2 * _NBUF)
            ),
        ))
    return _sc_reduce_cache[0](out, y)


def _tc_reduce_body(out_ref, y_ref, res_ref):
    pid = pl.program_id(0)
    d = out_ref[...] - y_ref[...]
    partial = jnp.sum(d * d, axis=1, keepdims=True)

    @pl.when(pid == 0)
    def _init():
        res_ref[...] = partial

    @pl.when(pid != 0)
    def _accum():
        res_ref[...] += partial


def _tc_reduce(out, y):
    nsteps = COLS // TC_CHUNK
    return pl.pallas_call(
        _tc_reduce_body,
        grid=(nsteps,),
        in_specs=[
            pl.BlockSpec((TC_ROWS, TC_CHUNK), lambda i: (0, i)),
            pl.BlockSpec((TC_ROWS, TC_CHUNK), lambda i: (0, i)),
        ],
        out_specs=pl.BlockSpec((TC_ROWS, 1), lambda i: (0, 0)),
        out_shape=jax.ShapeDtypeStruct((TC_ROWS, 1), jnp.float32),
        compiler_params=pltpu.CompilerParams(
            dimension_semantics=("arbitrary",),
        ),
    )(out, y)


def _merge_body(tc_ref, sc_ref, res_ref):
    vtc = tc_ref[...]
    vsc = jnp.sum(sc_ref[...], axis=1, keepdims=True)
    v = jnp.concatenate([vtc, vsc], axis=0) * (1.0 / COLS)
    eye = (lax.broadcasted_iota(jnp.int32, (ROWS, ROWS), 0) ==
           lax.broadcasted_iota(jnp.int32, (ROWS, ROWS), 1)).astype(jnp.float32)
    vrow = lax.dot_general(v, eye, (((0,), (0,)), ((), ())),
                           preferred_element_type=jnp.float32)
    gt = (vrow > v).astype(jnp.float32)
    ones = jnp.ones((ROWS, 1), jnp.float32)
    rank = lax.dot_general(gt, ones, (((1,), (0,)), ((), ())),
                           preferred_element_type=jnp.float32)
    cand = rank < K
    t = jnp.min(jnp.where(cand, v, jnp.inf))
    above = v > t
    n_above = jnp.sum(above.astype(jnp.float32))
    s_above = jnp.sum(jnp.where(above, v, 0.0))
    total = s_above + t * (K - n_above)
    res_ref[...] = total.reshape(1, 1)


def _merge(tc_part, sc_part):
    res = pl.pallas_call(
        _merge_body,
        out_shape=jax.ShapeDtypeStruct((1, 1), jnp.float32),
    )(tc_part, sc_part.reshape(SC_ROWS, _L))
    return res[0, 0]


def kernel(out, y):
    sc_part = _sc_reduce(out, y)
    tc_part = _tc_reduce(out, y)
    return _merge(tc_part, sc_part)


# speedup vs baseline: 2.9411x; 1.0064x over previous
import jax
import jax.numpy as jnp
from jax import lax
from jax.experimental import pallas as pl
from jax.experimental.pallas import tpu as pltpu
from jax.experimental.pallas import tpu_sc as plsc

ROWS = 128
COLS = 32768
K = 64

_NC = 2
_NS = 16
_L = 16
_NW = _NC * _NS

SC_ROWS = 32
TC_ROWS = ROWS - SC_ROWS
_SC_RPW = SC_ROWS // _NW
_SC_CHUNK = 8192
_SC_CPR = COLS // _SC_CHUNK
_SC_NCH = _SC_RPW * _SC_CPR
_UNROLL = 16
_NBUF = 4
TC_CHUNK = 4096


def _sc_reduce_body(out_hbm, y_hbm, res_hbm,
                    ob0, ob1, ob2, ob3, yb0, yb1, yb2, yb3, resbuf,
                    so0, so1, so2, so3, sy0, sy1, sy2, sy3):
    w = lax.axis_index("s") * _NC + lax.axis_index("c")
    obufs = (ob0, ob1, ob2, ob3)
    ybufs = (yb0, yb1, yb2, yb3)
    osems = (so0, so1, so2, so3)
    ysems = (sy0, sy1, sy2, sy3)

    def start(j):
        b = j % _NBUF
        row = TC_ROWS + w * _SC_RPW + j // _SC_CPR
        col = (j % _SC_CPR) * _SC_CHUNK
        oc = pltpu.async_copy(out_hbm.at[row, pl.ds(col, _SC_CHUNK)],
                              obufs[b], osems[b])
        yc = pltpu.async_copy(y_hbm.at[row, pl.ds(col, _SC_CHUNK)],
                              ybufs[b], ysems[b])
        return oc, yc

    pending = {}
    for j in range(min(_NBUF, _SC_NCH)):
        pending[j] = start(j)
    accs = tuple(jnp.zeros((_L,), jnp.float32) for _ in range(_UNROLL))
    for j in range(_SC_NCH):
        oc, yc = pending.pop(j)
        oc.wait()
        yc.wait()
        b = j % _NBUF
        ob, yb = obufs[b], ybufs[b]

        def body(i, a):
            base = i * (_L * _UNROLL)
            upd = []
            for u in range(_UNROLL):
                o = ob[pl.ds(base + u * _L, _L)]
                yv = yb[pl.ds(base + u * _L, _L)]
                d = o - yv
                upd.append(a[u] + d * d)
            return tuple(upd)

        accs = lax.fori_loop(0, _SC_CHUNK // (_L * _UNROLL), body, accs)
        if j + _NBUF < _SC_NCH:
            pending[j + _NBUF] = start(j + _NBUF)
        if j % _SC_CPR == _SC_CPR - 1:
            total = accs[0]
            for u in range(1, _UNROLL):
                total = total + accs[u]
            resbuf[j // _SC_CPR, :] = total
            accs = tuple(jnp.zeros((_L,), jnp.float32) for _ in range(_UNROLL))
    pltpu.sync_copy(resbuf, res_hbm.at[w])


_sc_reduce_cache = []


def _sc_reduce(out, y):
    if not _sc_reduce_cache:
        mesh = plsc.VectorSubcoreMesh(core_axis_name="c", subcore_axis_name="s")
        _sc_reduce_cache.append(pl.kernel(
            _sc_reduce_body,
            mesh=mesh,
            out_type=jax.ShapeDtypeStruct((_NW, _SC_RPW, _L), jnp.float32),
            scratch_types=(
                [pltpu.VMEM((_SC_CHUNK,), jnp.float32)] * (2 * _NBUF)
                + [pltpu.VMEM((_SC_RPW, _L), jnp.float32)]
                + [pltpu.SemaphoreType.DMA] * (2 * _NBUF)
            ),
        ))
    return _sc_reduce_cache[0](out, y)


def _tc_reduce_body(out_ref, y_ref, res_ref):
    pid = pl.program_id(0)
    d = out_ref[...] - y_ref[...]
    partial = jnp.sum(d * d, axis=1, keepdims=True)

    @pl.when(pid == 0)
    def _init():
        res_ref[...] = partial

    @pl.when(pid != 0)
    def _accum():
        res_ref[...] += partial


def _tc_reduce(out, y):
    nsteps = COLS // TC_CHUNK
    return pl.pallas_call(
        _tc_reduce_body,
        grid=(nsteps,),
        in_specs=[
            pl.BlockSpec((TC_ROWS, TC_CHUNK), lambda i: (0, i)),
            pl.BlockSpec((TC_ROWS, TC_CHUNK), lambda i: (0, i)),
        ],
        out_specs=pl.BlockSpec((TC_ROWS, 1), lambda i: (0, 0)),
        out_shape=jax.ShapeDtypeStruct((TC_ROWS, 1), jnp.float32),
        compiler_params=pltpu.CompilerParams(
            dimension_semantics=("arbitrary",),
        ),
    )(out, y)


def _merge_body(tc_ref, sc_ref, res_ref):
    vtc = tc_ref[...]
    vsc = jnp.sum(sc_ref[...], axis=1, keepdims=True)
    v = jnp.concatenate([vtc, vsc], axis=0) * (1.0 / COLS)
    eye = (lax.broadcasted_iota(jnp.int32, (ROWS, ROWS), 0) ==
           lax.broadcasted_iota(jnp.int32, (ROWS, ROWS), 1)).astype(jnp.float32)
    vrow = lax.dot_general(v, eye, (((0,), (0,)), ((), ())),
                           preferred_element_type=jnp.float32)
    gt = (vrow > v).astype(jnp.float32)
    ones = jnp.ones((ROWS, 1), jnp.float32)
    rank = lax.dot_general(gt, ones, (((1,), (0,)), ((), ())),
                           preferred_element_type=jnp.float32)
    cand = rank < K
    t = jnp.min(jnp.where(cand, v, jnp.inf))
    above = v > t
    n_above = jnp.sum(above.astype(jnp.float32))
    s_above = jnp.sum(jnp.where(above, v, 0.0))
    total = s_above + t * (K - n_above)
    res_ref[...] = total.reshape(1, 1)


def _merge(tc_part, sc_part):
    res = pl.pallas_call(
        _merge_body,
        out_shape=jax.ShapeDtypeStruct((1, 1), jnp.float32),
    )(tc_part, sc_part.reshape(SC_ROWS, _L))
    return res[0, 0]


def kernel(out, y):
    tc_part = _tc_reduce(out, y)
    sc_part = _sc_reduce(out, y)
    return _merge(tc_part, sc_part)


# speedup vs baseline: 6.9960x; 2.3787x over previous
import jax
import jax.numpy as jnp
from jax import lax
from jax.experimental import pallas as pl
from jax.experimental.pallas import tpu as pltpu

ROWS = 128
COLS = 32768
K = 64
CHUNK = 4096


def _body(out_ref, y_ref, res_ref, acc_ref):
    pid = pl.program_id(0)
    nsteps = pl.num_programs(0)

    d = out_ref[...] - y_ref[...]
    partial = jnp.sum(d * d, axis=1, keepdims=True)

    @pl.when(pid == 0)
    def _init():
        acc_ref[...] = partial

    @pl.when(pid != 0)
    def _accum():
        acc_ref[...] += partial

    @pl.when(pid == nsteps - 1)
    def _finish():
        v = acc_ref[...] * (1.0 / COLS)
        eye = (lax.broadcasted_iota(jnp.int32, (ROWS, ROWS), 0) ==
               lax.broadcasted_iota(jnp.int32, (ROWS, ROWS), 1)
               ).astype(jnp.float32)
        vrow = lax.dot_general(v, eye, (((0,), (0,)), ((), ())),
                               preferred_element_type=jnp.float32)
        gt = (vrow > v).astype(jnp.float32)
        ones = jnp.ones((ROWS, 1), jnp.float32)
        rank = lax.dot_general(gt, ones, (((1,), (0,)), ((), ())),
                               preferred_element_type=jnp.float32)
        cand = rank < K
        t = jnp.min(jnp.where(cand, v, jnp.inf))
        above = v > t
        n_above = jnp.sum(above.astype(jnp.float32))
        s_above = jnp.sum(jnp.where(above, v, 0.0))
        total = s_above + t * (K - n_above)
        res_ref[...] = total.reshape(1, 1)


def kernel(out, y):
    nsteps = COLS // CHUNK
    res = pl.pallas_call(
        _body,
        grid=(nsteps,),
        in_specs=[
            pl.BlockSpec((ROWS, CHUNK), lambda i: (0, i)),
            pl.BlockSpec((ROWS, CHUNK), lambda i: (0, i)),
        ],
        out_specs=pl.BlockSpec((1, 1), lambda i: (0, 0)),
        out_shape=jax.ShapeDtypeStruct((1, 1), jnp.float32),
        scratch_shapes=[pltpu.VMEM((ROWS, 1), jnp.float32)],
        compiler_params=pltpu.CompilerParams(
            dimension_semantics=("arbitrary",),
        ),
    )(out, y)
    return res[0, 0]


# speedup vs baseline: 7.3128x; 1.0453x over previous
import jax
import jax.numpy as jnp
from jax import lax
from jax.experimental import pallas as pl
from jax.experimental.pallas import tpu as pltpu

ROWS = 128
COLS = 32768
K = 64
CHUNK = 8192


def _body(out_ref, y_ref, res_ref, acc_ref):
    pid = pl.program_id(0)
    nsteps = pl.num_programs(0)

    d = out_ref[...] - y_ref[...]
    partial = jnp.sum(d * d, axis=1, keepdims=True)

    @pl.when(pid == 0)
    def _init():
        acc_ref[...] = partial

    @pl.when(pid != 0)
    def _accum():
        acc_ref[...] += partial

    @pl.when(pid == nsteps - 1)
    def _finish():
        v = acc_ref[...] * (1.0 / COLS)
        eye = (lax.broadcasted_iota(jnp.int32, (ROWS, ROWS), 0) ==
               lax.broadcasted_iota(jnp.int32, (ROWS, ROWS), 1)
               ).astype(jnp.float32)
        vrow = lax.dot_general(v, eye, (((0,), (0,)), ((), ())),
                               preferred_element_type=jnp.float32)
        gt = (vrow > v).astype(jnp.float32)
        ones = jnp.ones((ROWS, 1), jnp.float32)
        rank = lax.dot_general(gt, ones, (((1,), (0,)), ((), ())),
                               preferred_element_type=jnp.float32)
        cand = rank < K
        t = jnp.min(jnp.where(cand, v, jnp.inf))
        above = v > t
        n_above = jnp.sum(above.astype(jnp.float32))
        s_above = jnp.sum(jnp.where(above, v, 0.0))
        total = s_above + t * (K - n_above)
        res_ref[...] = total.reshape(1, 1)


def kernel(out, y):
    nsteps = COLS // CHUNK
    res = pl.pallas_call(
        _body,
        grid=(nsteps,),
        in_specs=[
            pl.BlockSpec((ROWS, CHUNK), lambda i: (0, i)),
            pl.BlockSpec((ROWS, CHUNK), lambda i: (0, i)),
        ],
        out_specs=pl.BlockSpec((1, 1), lambda i: (0, 0)),
        out_shape=jax.ShapeDtypeStruct((1, 1), jnp.float32),
        scratch_shapes=[pltpu.VMEM((ROWS, 1), jnp.float32)],
        compiler_params=pltpu.CompilerParams(
            dimension_semantics=("arbitrary",),
        ),
    )(out, y)
    return res[0, 0]
